# baseline (speedup 1.0000x reference)
import functools

import jax
import jax.numpy as jnp
from jax import lax
from jax.experimental import pallas as pl
from jax.experimental.pallas import tpu as pltpu
from jax.experimental.pallas import tpu_sc as plsc

_NC = 2
_NS = 16
_NW = _NC * _NS
_L = 16

_LN2 = 0.6931471805599453
_C5 = 0.030449004538686555
_C4 = -0.13158182508879562
_C3 = 0.28527268109059173
_C2 = -0.49023072342341184
_C1 = 0.9992354838332752
_C0 = 9.975032552169407e-06


def _softlog(x):
    bits = lax.bitcast_convert_type(x, jnp.int32)
    e = (bits >> 23) - 127
    mant = lax.bitcast_convert_type((bits & 0x007FFFFF) | 0x3F800000,
                                    jnp.float32)
    t = mant - 1.0
    p = ((((_C5 * t + _C4) * t + _C3) * t + _C2) * t + _C1) * t + _C0
    return e.astype(jnp.float32) * _LN2 + p


def _lane_gather(v, idx):
    dnums = lax.GatherDimensionNumbers(
        offset_dims=(), collapsed_slice_dims=(0,), start_index_map=(0,))
    return lax.gather(v, idx[:, None], dnums, (1,),
                      mode=lax.GatherScatterMode.PROMISE_IN_BOUNDS)


def _sc_partials_body(yp_hbm, yt_hbm, mi_hbm, out_hbm, yp_v, yt_v, mi_v, acc_v):
    chunk = yp_v.shape[0]
    ngrp = chunk // (4 * _L)
    wid = lax.axis_index("s") * _NC + lax.axis_index("c")
    pltpu.sync_copy(yp_hbm.at[pl.ds(wid * chunk, chunk)], yp_v)
    pltpu.sync_copy(yt_hbm.at[pl.ds(wid * chunk, chunk)], yt_v)
    pltpu.sync_copy(mi_hbm.at[pl.ds(wid * (chunk // 4), chunk // 4)], mi_v)

    iota = lax.iota(jnp.int32, _L)
    shv = (iota & 3) * 8
    idx = [(iota >> 2) + 4 * j for j in range(4)]
    zf = jnp.zeros((_L,), jnp.float32)

    def body(g, carry):
        n, sp, st, spp, stt, spt, slt = carry
        mw = mi_v[pl.ds(g * _L, _L)]
        for j in range(4):
            m01 = (_lane_gather(mw, idx[j]) >> shv) & 1
            b = m01 != 0
            off = (g * 4 + j) * _L
            p = yp_v[pl.ds(off, _L)]
            t = yt_v[pl.ds(off, _L)]
            pm = jnp.where(b, p, zf)
            tm = jnp.where(b, t, zf)
            lg = _softlog(p + 1e-8)
            n = n + m01
            sp = sp + pm
            st = st + tm
            spp = spp + p * pm
            stt = stt + t * tm
            spt = spt + p * tm
            slt = slt + lg * tm
        return (n, sp, st, spp, stt, spt, slt)

    zi = jnp.zeros((_L,), jnp.int32)
    accs = lax.fori_loop(0, ngrp, body, (zi,) + (zf,) * 6)
    acc_v[pl.ds(0, _L)] = accs[0].astype(jnp.float32)
    for j in range(1, 7):
        acc_v[pl.ds(j * _L, _L)] = accs[j]
    pltpu.sync_copy(acc_v, out_hbm.at[wid])


def _sc_partials(yp, yt, mi):
    chunk = yp.shape[0] // _NW
    fn = functools.partial(
        pl.kernel,
        mesh=plsc.VectorSubcoreMesh(core_axis_name="c", subcore_axis_name="s"),
        out_type=jax.ShapeDtypeStruct((_NW, 7 * _L), jnp.float32),
        scratch_types=[
            pltpu.VMEM((chunk,), jnp.float32),
            pltpu.VMEM((chunk,), jnp.float32),
            pltpu.VMEM((chunk // 4,), jnp.int32),
            pltpu.VMEM((7 * _L,), jnp.float32),
        ],
    )(_sc_partials_body)
    return fn(yp, yt, mi)


def _tc_finalize_body(ts_ref, parts_ref, out_ref):
    parts = parts_ref[...]
    s = [jnp.sum(parts[:, j * _L:(j + 1) * _L]) for j in range(7)]
    n, sp, st, spp, stt, spt, slt = s
    pois = sp - slt
    eps = 1e-6
    mx = sp / n
    my = st / n
    num = spt - n * mx * my
    nx = jnp.sqrt(jnp.maximum(spp - n * mx * mx, 0.0))
    ny = jnp.sqrt(jnp.maximum(stt - n * my * my, 0.0))
    cos = num / (jnp.maximum(nx, eps) * jnp.maximum(ny, eps))
    w = jnp.maximum(0.0, 1.0 - ts_ref[0, 0] / 10.0)
    out_ref[...] = jnp.full((1, 1), (1.0 - cos) + w * (pois / n), jnp.float32)


def _tc_finalize(ts, parts):
    return pl.pallas_call(
        _tc_finalize_body,
        out_shape=jax.ShapeDtypeStruct((1, 1), jnp.float32),
        in_specs=[
            pl.BlockSpec(memory_space=pltpu.SMEM),
            pl.BlockSpec(memory_space=pltpu.VMEM),
        ],
        out_specs=pl.BlockSpec(memory_space=pltpu.VMEM),
    )(ts, parts)


def kernel(y_pred, y_true, mask, timestamp):
    yp = y_pred.reshape(-1)
    yt = y_true.reshape(-1)
    mi = lax.bitcast_convert_type(
        mask.reshape(-1, 4).astype(jnp.uint8), jnp.int32)
    parts = _sc_partials(yp, yt, mi)
    ts = jnp.asarray(timestamp, jnp.float32).reshape(1, 1)
    return _tc_finalize(ts, parts).reshape(())


# speedup vs baseline: 1.6693x; 1.6693x over previous
import functools

import jax
import jax.numpy as jnp
from jax import lax
from jax.experimental import pallas as pl
from jax.experimental.pallas import tpu as pltpu
from jax.experimental.pallas import tpu_sc as plsc

_NC = 2
_NS = 16
_NW = _NC * _NS
_L = 16
_UNROLL = 4

_LN2 = 0.6931471805599453
_C5 = 0.030449004538686555
_C4 = -0.13158182508879562
_C3 = 0.28527268109059173
_C2 = -0.49023072342341184
_C1 = 0.9992354838332752
_C0 = 9.975032552169407e-06


def _softlog(x):
    bits = lax.bitcast_convert_type(x, jnp.int32)
    e = (bits >> 23) - 127
    mant = lax.bitcast_convert_type((bits & 0x007FFFFF) | 0x3F800000,
                                    jnp.float32)
    t = mant - 1.0
    p = ((((_C5 * t + _C4) * t + _C3) * t + _C2) * t + _C1) * t + _C0
    return e.astype(jnp.float32) * _LN2 + p


def _sc_partials_body(yp_hbm, yt_hbm, mf_hbm, out_hbm,
                      yp_v, yt_v, mf_v, acc_v, sem):
    chunk = yp_v.shape[0]
    ngrp = chunk // (_UNROLL * _L)
    wid = lax.axis_index("s") * _NC + lax.axis_index("c")
    base = wid * chunk
    cp0 = pltpu.async_copy(yp_hbm.at[pl.ds(base, chunk)], yp_v, sem)
    cp1 = pltpu.async_copy(yt_hbm.at[pl.ds(base, chunk)], yt_v, sem)
    cp2 = pltpu.async_copy(mf_hbm.at[pl.ds(base, chunk)], mf_v, sem)
    cp0.wait()
    cp1.wait()
    cp2.wait()

    def body(g, carry):
        n, sp, st, spp, stt, spt, slt = carry
        for j in range(_UNROLL):
            off = (g * _UNROLL + j) * _L
            p = yp_v[pl.ds(off, _L)]
            t = yt_v[pl.ds(off, _L)]
            m = mf_v[pl.ds(off, _L)]
            pm = p * m
            tm = t * m
            lg = _softlog(p + 1e-8)
            n = n + m
            sp = sp + pm
            st = st + tm
            spp = spp + p * pm
            stt = stt + t * tm
            spt = spt + p * tm
            slt = slt + lg * tm
        return (n, sp, st, spp, stt, spt, slt)

    zf = jnp.zeros((_L,), jnp.float32)
    accs = lax.fori_loop(0, ngrp, body, (zf,) * 7)
    for j in range(7):
        acc_v[pl.ds(j * _L, _L)] = accs[j]
    pltpu.sync_copy(acc_v, out_hbm.at[wid])


def _sc_partials(yp, yt, mf):
    chunk = yp.shape[0] // _NW
    fn = functools.partial(
        pl.kernel,
        mesh=plsc.VectorSubcoreMesh(core_axis_name="c", subcore_axis_name="s"),
        out_type=jax.ShapeDtypeStruct((_NW, 7 * _L), jnp.float32),
        scratch_types=[
            pltpu.VMEM((chunk,), jnp.float32),
            pltpu.VMEM((chunk,), jnp.float32),
            pltpu.VMEM((chunk,), jnp.float32),
            pltpu.VMEM((7 * _L,), jnp.float32),
            pltpu.SemaphoreType.DMA,
        ],
    )(_sc_partials_body)
    return fn(yp, yt, mf)


def _tc_finalize_body(ts_ref, parts_ref, out_ref):
    parts = parts_ref[...]
    s = [jnp.sum(parts[:, j * _L:(j + 1) * _L]) for j in range(7)]
    n, sp, st, spp, stt, spt, slt = s
    pois = sp - slt
    eps = 1e-6
    mx = sp / n
    my = st / n
    num = spt - n * mx * my
    nx = jnp.sqrt(jnp.maximum(spp - n * mx * mx, 0.0))
    ny = jnp.sqrt(jnp.maximum(stt - n * my * my, 0.0))
    cos = num / (jnp.maximum(nx, eps) * jnp.maximum(ny, eps))
    w = jnp.maximum(0.0, 1.0 - ts_ref[0, 0] / 10.0)
    out_ref[...] = jnp.full((1, 1), (1.0 - cos) + w * (pois / n), jnp.float32)


def _tc_finalize(ts, parts):
    return pl.pallas_call(
        _tc_finalize_body,
        out_shape=jax.ShapeDtypeStruct((1, 1), jnp.float32),
        in_specs=[
            pl.BlockSpec(memory_space=pltpu.SMEM),
            pl.BlockSpec(memory_space=pltpu.VMEM),
        ],
        out_specs=pl.BlockSpec(memory_space=pltpu.VMEM),
    )(ts, parts)


def kernel(y_pred, y_true, mask, timestamp):
    yp = y_pred.reshape(-1)
    yt = y_true.reshape(-1)
    mf = mask.reshape(-1).astype(jnp.float32)
    parts = _sc_partials(yp, yt, mf)
    ts = jnp.asarray(timestamp, jnp.float32).reshape(1, 1)
    return _tc_finalize(ts, parts).reshape(())
